# Initial kernel scaffold; baseline (speedup 1.0000x reference)
import jax
import jax.numpy as jnp
from jax.experimental import pallas as pl


def kernel(energy, mask):
    raise NotImplementedError("write your pallas kernel here")


# speedup vs baseline: 14.0876x; 14.0876x over previous
import jax
import jax.numpy as jnp
from jax import lax
from jax.experimental import pallas as pl
from jax.experimental.pallas import tpu as pltpu

_TEMPERATURE = 1.0
_TARGET_RATE = 0.05
_BUDGET_WEIGHT = 1.0
_SPARSE_THRESHOLD = 0.5
_SPARSE_TEMPERATURE = 0.25
_TOP_K = 1024
_EPS = 1e-06

_INT_MIN = -2147483648
_INT_MAX = 2147483647

_ROW_BLOCK = 32


def _monokey(e):
    b = lax.bitcast_convert_type(e, jnp.int32)
    flip = lax.shift_right_arithmetic(b, 31) & jnp.int32(0x7FFFFFFF)
    return b ^ flip


def _mid(lo, hi):
    diff = hi - lo
    half = lax.shift_right_logical(diff, 1)
    return lo + jnp.maximum(half, jnp.int32(1))


def _body(energy_ref, mask_ref, act_ref, tkmask_ref, gmass_ref,
          bloss_ref, ent_ref, arate_ref, acc_ref):
    step = pl.program_id(0)
    n_steps = pl.num_programs(0)

    e = energy_ref[...]
    valid = mask_ref[...] != 0
    rows, cols = e.shape
    k = jnp.int32(_TOP_K)

    key = jnp.where(valid, _monokey(e), _INT_MIN)

    def value_step(_, carry):
        lo, hi = carry
        mid = _mid(lo, hi)
        cnt = jnp.sum((key >= mid).astype(jnp.int32), axis=1, keepdims=True)
        take = cnt >= k
        return jnp.where(take, mid, lo), jnp.where(take, hi, mid)

    lo0 = jnp.full((rows, 1), _INT_MIN, jnp.int32)
    hi0 = jnp.full((rows, 1), _INT_MAX, jnp.int32)
    thr, _ = lax.fori_loop(0, 32, value_step, (lo0, hi0))

    above = key > thr
    tie = key == thr
    n_gt = jnp.sum(above.astype(jnp.int32), axis=1, keepdims=True)
    need = k - n_gt

    col = lax.broadcasted_iota(jnp.int32, (rows, cols), 1)

    def index_step(_, carry):
        lo, hi = carry
        mid = _mid(lo, hi)
        cnt = jnp.sum((tie & (col < mid)).astype(jnp.int32), axis=1,
                      keepdims=True)
        enough = cnt >= need
        return jnp.where(enough, lo, mid), jnp.where(enough, mid, hi)

    ilo0 = jnp.zeros((rows, 1), jnp.int32)
    ihi0 = jnp.full((rows, 1), jnp.int32(cols + 1), jnp.int32)
    _, cut = lax.fori_loop(0, 16, index_step, (ilo0, ihi0))

    selected = (above | (tie & (col < cut))) & valid

    a = jax.nn.sigmoid(e / _TEMPERATURE)
    gate = jax.nn.sigmoid((a - _SPARSE_THRESHOLD) / _SPARSE_TEMPERATURE)
    act = jnp.where(selected, a * gate, 0.0)

    act_ref[...] = act
    tkmask_ref[...] = selected
    gmass_ref[...] = jnp.sum(act, axis=1, keepdims=True)

    validf = valid.astype(jnp.float32)
    part_valid = jnp.sum(validf)
    part_act = jnp.sum(act)
    p = jnp.clip(act, _EPS, 1.0 - _EPS)
    ent_vals = -(p * jnp.log(p) + (1.0 - p) * jnp.log(1.0 - p))
    part_ent = jnp.sum(ent_vals * validf)

    @pl.when(step == 0)
    def _init():
        acc_ref[0] = part_valid
        acc_ref[1] = part_act
        acc_ref[2] = part_ent

    @pl.when(step != 0)
    def _accum():
        acc_ref[0] += part_valid
        acc_ref[1] += part_act
        acc_ref[2] += part_ent

    @pl.when(step == n_steps - 1)
    def _finalize():
        valid_count = jnp.maximum(acc_ref[0], 1.0)
        active_rate = acc_ref[1] / valid_count
        arate_ref[0, 0] = active_rate
        ent_ref[0, 0] = acc_ref[2] / valid_count
        bloss_ref[0, 0] = _BUDGET_WEIGHT * jnp.square(
            active_rate - jnp.float32(_TARGET_RATE))


@jax.jit
def kernel(energy, mask):
    n_rows, n_cols = energy.shape
    energy = energy.astype(jnp.float32)
    mask_i8 = mask.astype(jnp.int8)

    grid = (n_rows // _ROW_BLOCK,)
    out_shapes = (
        jax.ShapeDtypeStruct((n_rows, n_cols), jnp.float32),
        jax.ShapeDtypeStruct((n_rows, n_cols), jnp.bool_),
        jax.ShapeDtypeStruct((n_rows, 1), jnp.float32),
        jax.ShapeDtypeStruct((1, 1), jnp.float32),
        jax.ShapeDtypeStruct((1, 1), jnp.float32),
        jax.ShapeDtypeStruct((1, 1), jnp.float32),
    )
    row_spec = pl.BlockSpec((_ROW_BLOCK, n_cols), lambda i: (i, 0))
    scalar_spec = pl.BlockSpec(memory_space=pltpu.SMEM)
    act, tkmask, gmass, bloss, ent, arate = pl.pallas_call(
        _body,
        grid=grid,
        in_specs=[row_spec, row_spec],
        out_specs=(
            row_spec,
            row_spec,
            pl.BlockSpec((_ROW_BLOCK, 1), lambda i: (i, 0)),
            scalar_spec,
            scalar_spec,
            scalar_spec,
        ),
        out_shape=out_shapes,
        scratch_shapes=[pltpu.SMEM((3,), jnp.float32)],
    )(energy, mask_i8)

    return (act, act, bloss[0, 0], ent[0, 0], arate[0, 0], tkmask,
            gmass[:, 0])
